# Initial kernel scaffold; baseline (speedup 1.0000x reference)
import jax
import jax.numpy as jnp
from jax.experimental import pallas as pl


def kernel(action_idx, table):
    raise NotImplementedError("write your pallas kernel here")


# speedup vs baseline: 1.0217x; 1.0217x over previous
import functools

import jax
import jax.numpy as jnp
from jax import lax
from jax.experimental import pallas as pl
from jax.experimental.pallas import tpu as pltpu
from jax.experimental.pallas import tpu_sc as plsc

BATCH = 16384
HIST = 50
EMBED = 32
B = BATCH * HIST
NC = 2
NS = 16
NW = NC * NS
PER_W = B // NW
CHUNK = 128
N_CH = PER_W // CHUNK

_mesh = plsc.VectorSubcoreMesh(
    core_axis_name="c", subcore_axis_name="s", num_cores=NC, num_subcores=NS
)


@functools.partial(
    pl.kernel,
    out_type=jax.ShapeDtypeStruct((B, EMBED), jnp.float32),
    mesh=_mesh,
    scratch_types=[
        pltpu.VMEM((N_CH, CHUNK), jnp.int32),
        pltpu.VMEM((CHUNK, EMBED), jnp.float32),
        pltpu.SemaphoreType.DMA,
    ],
    compiler_params=pltpu.CompilerParams(use_tc_tiling_on_sc=False),
)
def _gather_kernel(idx_hbm, table_hbm, out_hbm, idx_v, rows_v, sem):
    wid = lax.axis_index("s") * NC + lax.axis_index("c")
    pltpu.sync_copy(idx_hbm.at[wid], idx_v)
    base = wid * PER_W

    @pl.loop(0, N_CH)
    def _chunk(g):
        pltpu.async_copy(table_hbm.at[idx_v.at[g]], rows_v, sem).wait()
        pltpu.sync_copy(rows_v, out_hbm.at[pl.ds(base + g * CHUNK, CHUNK)])


def kernel(action_idx, table):
    idx = action_idx.reshape(-1).astype(jnp.int32).reshape(NW, N_CH, CHUNK)
    out = _gather_kernel(idx, table)
    return out.reshape(BATCH, HIST, EMBED)


# speedup vs baseline: 1.1113x; 1.0877x over previous
import functools

import jax
import jax.numpy as jnp
from jax import lax
from jax.experimental import pallas as pl
from jax.experimental.pallas import tpu as pltpu
from jax.experimental.pallas import tpu_sc as plsc

BATCH = 16384
HIST = 50
EMBED = 32
B = BATCH * HIST
NC = 2
NS = 16
NW = NC * NS
PER_W = B // NW
CHUNK = 128
N_CH = PER_W // CHUNK
NBUF = 8
N_GRP = N_CH // NBUF

_mesh = plsc.VectorSubcoreMesh(
    core_axis_name="c", subcore_axis_name="s", num_cores=NC, num_subcores=NS
)


@functools.partial(
    pl.kernel,
    out_type=jax.ShapeDtypeStruct((B, EMBED), jnp.float32),
    mesh=_mesh,
    scratch_types=[
        pltpu.VMEM((N_CH, CHUNK), jnp.int32),
        pltpu.VMEM((NBUF, CHUNK, EMBED), jnp.float32),
        [pltpu.SemaphoreType.DMA] * NBUF,
        [pltpu.SemaphoreType.DMA] * NBUF,
    ],
    compiler_params=pltpu.CompilerParams(use_tc_tiling_on_sc=False),
)
def _gather_kernel(idx_hbm, table_hbm, out_hbm, idx_v, rows_v, gsems, wsems):
    wid = lax.axis_index("s") * NC + lax.axis_index("c")
    pltpu.sync_copy(idx_hbm.at[wid], idx_v)
    base = wid * PER_W

    def gather(c, b):
        return pltpu.make_async_copy(
            table_hbm.at[idx_v.at[c]], rows_v.at[b], gsems[b]
        )

    def write(c, b):
        return pltpu.make_async_copy(
            rows_v.at[b], out_hbm.at[pl.ds(base + c * CHUNK, CHUNK)], wsems[b]
        )

    for b in range(NBUF):
        gather(b, b).start()

    @pl.loop(0, N_GRP - 1)
    def _grp(g):
        g0 = g * NBUF
        for b in range(NBUF):
            gather(g0 + b, b).wait()
            write(g0 + b, b).start()
        for b in range(NBUF):
            write(g0 + b, b).wait()
            gather(g0 + NBUF + b, b).start()

    g0 = (N_GRP - 1) * NBUF
    for b in range(NBUF):
        gather(g0 + b, b).wait()
        write(g0 + b, b).start()
    for b in range(NBUF):
        write(g0 + b, b).wait()


def kernel(action_idx, table):
    idx = action_idx.reshape(-1).astype(jnp.int32).reshape(NW, N_CH, CHUNK)
    out = _gather_kernel(idx, table)
    return out.reshape(BATCH, HIST, EMBED)


# speedup vs baseline: 1.7720x; 1.5945x over previous
import functools

import jax
import jax.numpy as jnp
from jax import lax
from jax.experimental import pallas as pl
from jax.experimental.pallas import tpu as pltpu
from jax.experimental.pallas import tpu_sc as plsc

BATCH = 16384
HIST = 50
EMBED = 32
NC = 2
NS = 16
NW = NC * NS
PER_W = BATCH // NW
NBUF = 8
N_GRP = PER_W // NBUF

_mesh = plsc.VectorSubcoreMesh(
    core_axis_name="c", subcore_axis_name="s", num_cores=NC, num_subcores=NS
)


@functools.partial(
    pl.kernel,
    out_type=jax.ShapeDtypeStruct((BATCH, HIST, EMBED), jnp.float32),
    mesh=_mesh,
    scratch_types=[
        pltpu.VMEM((PER_W, HIST), jnp.int32),
        pltpu.VMEM((NBUF, HIST, EMBED), jnp.float32),
        [pltpu.SemaphoreType.DMA] * NBUF,
        [pltpu.SemaphoreType.DMA] * NBUF,
    ],
    compiler_params=pltpu.CompilerParams(use_tc_tiling_on_sc=False),
)
def _gather_kernel(idx_hbm, table_hbm, out_hbm, idx_v, rows_v, gsems, wsems):
    wid = lax.axis_index("s") * NC + lax.axis_index("c")
    base = wid * PER_W
    pltpu.sync_copy(idx_hbm.at[pl.ds(base, PER_W)], idx_v)

    def gather(c, b):
        return pltpu.make_async_copy(
            table_hbm.at[idx_v.at[c]], rows_v.at[b], gsems[b]
        )

    def write(c, b):
        return pltpu.make_async_copy(
            rows_v.at[b], out_hbm.at[base + c], wsems[b]
        )

    for b in range(NBUF):
        gather(b, b).start()

    @pl.loop(0, N_GRP - 1)
    def _grp(g):
        g0 = g * NBUF
        for b in range(NBUF):
            gather(g0 + b, b).wait()
            write(g0 + b, b).start()
        for b in range(NBUF):
            write(g0 + b, b).wait()
            gather(g0 + NBUF + b, b).start()

    g0 = (N_GRP - 1) * NBUF
    for b in range(NBUF):
        gather(g0 + b, b).wait()
        write(g0 + b, b).start()
    for b in range(NBUF):
        write(g0 + b, b).wait()


def kernel(action_idx, table):
    return _gather_kernel(action_idx.astype(jnp.int32), table)


# speedup vs baseline: 5.0721x; 2.8624x over previous
import functools

import jax
import jax.numpy as jnp
from jax import lax
from jax.experimental import pallas as pl
from jax.experimental.pallas import tpu as pltpu
from jax.experimental.pallas import tpu_sc as plsc

BATCH = 16384
HIST = 50
EMBED = 32
NC = 2
NS = 16
NW = NC * NS
BT = BATCH // 128
BT_W = BT // NW
N_UNIT = HIST * BT_W
NBUF = 2
N_GRP = N_UNIT // NBUF

_mesh = plsc.VectorSubcoreMesh(
    core_axis_name="c", subcore_axis_name="s", num_cores=NC, num_subcores=NS
)


@functools.partial(
    pl.kernel,
    out_type=jax.ShapeDtypeStruct((HIST, EMBED // 8, BT, 8, 128), jnp.float32),
    mesh=_mesh,
    scratch_types=[
        pltpu.VMEM((HIST, 128 * BT_W), jnp.int32),
        pltpu.VMEM((NBUF, 128, EMBED), jnp.float32),
        pltpu.VMEM((NBUF, EMBED, 128), jnp.float32),
        [pltpu.SemaphoreType.DMA] * NBUF,
        [pltpu.SemaphoreType.DMA] * NBUF,
    ],
    compiler_params=pltpu.CompilerParams(
        use_tc_tiling_on_sc=False, needs_layout_passes=False
    ),
)
def _gather_kernel(idx_hbm, table_hbm, out_hbm, idx_v, rows_v, tile_v,
                   gsems, wsems):
    wid = lax.axis_index("s") * NC + lax.axis_index("c")
    pltpu.sync_copy(idx_hbm.at[:, pl.ds(wid * (128 * BT_W), 128 * BT_W)],
                    idx_v)

    def gather(u, b):
        return pltpu.make_async_copy(
            table_hbm.at[idx_v.at[u // BT_W, pl.ds((u % BT_W) * 128, 128)]],
            rows_v.at[b], gsems[b])

    def write(u, b):
        h, bt = u // BT_W, wid * BT_W + u % BT_W
        return [
            pltpu.make_async_copy(
                tile_v.at[b, pl.ds(ct * 8, 8), :], out_hbm.at[h, ct, bt],
                wsems[b])
            for ct in range(EMBED // 8)
        ]

    def transpose(b):
        lanes = lax.iota(jnp.int32, 16)
        shifts = [(lanes + k) % 16 for k in range(16)]

        @plsc.parallel_loop(0, 128, 16, unroll=2)
        def _bc(bc0):
            row_idx = lanes + bc0
            for c0 in range(0, EMBED, 16):
                for k in range(16):
                    col_idx = shifts[k] + c0
                    vals = plsc.load_gather(rows_v.at[b], [row_idx, col_idx])
                    plsc.store_scatter(tile_v.at[b], [col_idx, row_idx], vals)

    for b in range(NBUF):
        gather(b, b).start()

    @pl.loop(0, N_GRP)
    def _grp(g):
        g0 = g * NBUF
        for b in range(NBUF):
            u = g0 + b
            gather(u, b).wait()

            @pl.when(g > 0)
            def _wait_prev_write():
                for d in write(u - NBUF, b):
                    d.wait()

            transpose(b)
            for d in write(u, b):
                d.start()

            @pl.when(g < N_GRP - 1)
            def _refill():
                gather(u + NBUF, b).start()

    g0 = (N_GRP - 1) * NBUF
    for b in range(NBUF):
        for d in write(g0 + b, b):
            d.wait()


NUM_ROWS = 1000000
N_CKF = NUM_ROWS // 128
N_K = N_CKF // NW + 1
NBUF_A = 2


@functools.partial(
    pl.kernel,
    out_type=jax.ShapeDtypeStruct((NUM_ROWS * EMBED,), jnp.float32),
    mesh=_mesh,
    scratch_types=[
        [pltpu.VMEM((EMBED, 128), jnp.float32)] * NBUF_A,
        [pltpu.VMEM((128 * EMBED,), jnp.float32)] * NBUF_A,
        [pltpu.SemaphoreType.DMA] * NBUF_A,
        [pltpu.SemaphoreType.DMA] * NBUF_A,
    ],
    compiler_params=pltpu.CompilerParams(
        use_tc_tiling_on_sc=True, needs_layout_passes=False
    ),
)
def _detile_kernel(tab_t, out_hbm, slab_v, lin_v, gsems, wsems):
    wid = lax.axis_index("s") * NC + lax.axis_index("c")
    n = jnp.where(wid < N_CKF - (N_CKF // NW) * NW,
                  N_CKF // NW + 1, N_CKF // NW)

    def rd_op(k, b, start):
        j = wid + k * NW
        d = pltpu.make_async_copy(
            tab_t.at[:, pl.ds(j * 128, 128)], slab_v[b], gsems[b])
        d.start() if start else d.wait()

    def wr_op(k, b, start):
        j = wid + k * NW
        d = pltpu.make_async_copy(
            lin_v[b], out_hbm.at[pl.ds(j * (128 * EMBED), 128 * EMBED)],
            wsems[b])
        d.start() if start else d.wait()

    def transpose_a(b):
        lanes = lax.iota(jnp.int32, 16)
        shifts = [(lanes + k) % 16 for k in range(16)]

        @plsc.parallel_loop(0, 128, 16, unroll=2)
        def _ii(ii0):
            for c0 in range(0, EMBED, 16):
                c_idx = lanes + c0
                for k in range(16):
                    i_idx = shifts[k] + ii0
                    vals = plsc.load_gather(slab_v[b], [c_idx, i_idx])
                    plsc.store_scatter(
                        lin_v[b], [i_idx * EMBED + c_idx], vals)

    for b in range(NBUF_A):
        rd_op(b, b, True)

    @pl.loop(0, (N_K + 1) // NBUF_A)
    def _grp(g):
        for b in range(NBUF_A):
            k = g * NBUF_A + b

            @pl.when(k < n)
            def _do():
                rd_op(k, b, False)

                @pl.when(k >= NBUF_A)
                def _wprev():
                    wr_op(k - NBUF_A, b, False)

                transpose_a(b)
                wr_op(k, b, True)

                @pl.when(k + NBUF_A < n)
                def _refill():
                    rd_op(k + NBUF_A, b, True)

    for b in range(NBUF_A):
        k_last = n - 1 - ((n - 1 - b) % NBUF_A)
        wr_op(k_last, b, False)


def kernel(action_idx, table):
    tab_lin = _detile_kernel(table.T)
    tail = lax.slice(table, (N_CKF * 128, 0), (NUM_ROWS, EMBED)).reshape(-1)
    tab_full = lax.dynamic_update_slice(tab_lin, tail, (N_CKF * 128 * EMBED,))
    idx_t = action_idx.astype(jnp.int32).T
    out5 = _gather_kernel(idx_t, tab_full.reshape(NUM_ROWS, EMBED))
    return lax.transpose(out5, (2, 4, 0, 1, 3)).reshape(BATCH, HIST, EMBED)
